# baseline (speedup 1.0000x reference)
import functools

import jax
import jax.numpy as jnp
from jax import lax
from jax.experimental import pallas as pl
from jax.experimental.pallas import tpu as pltpu
from jax.experimental.pallas import tpu_sc as plsc

N = 10000
E = 320000
F_IN = 128
H = 64
SEQ = 24

NC, NS, L = 2, 16, 16
NPAD = 10240
C = 80
ROWS = E // C
ROWS_FULL = ROWS // NS
ROWS_HALF = ROWS // (NC * NS)
NROW_TILE = N // NS
WB = 125

_f32 = jnp.float32


def _newton_rsqrt(x):
    xi = lax.bitcast_convert_type(x, jnp.int32)
    yi = jnp.int32(0x5F3759DF) - lax.shift_right_logical(xi, 1)
    y = lax.bitcast_convert_type(yi, _f32)
    for _ in range(3):
        y = y * (1.5 - 0.5 * x * y * y)
    return jnp.where(x > 0, y, 0.0)


def _norm_body(src2, dst2, w2, norm_out, sidx, didx, wbuf, nbuf, zbuf,
               dinv_v, deg_sh, sem):
    c = lax.axis_index("c")
    s = lax.axis_index("s")
    del sem

    @pl.loop(0, NPAD // NS // 16)
    def _z(i):
        zbuf[pl.ds(i * 16, 16)] = jnp.zeros((16,), _f32)
    pltpu.sync_copy(zbuf, deg_sh.at[pl.ds(s * (NPAD // NS), NPAD // NS)])
    plsc.subcore_barrier()

    @pl.loop(0, ROWS_FULL)
    def _deg(j):
        r = s * ROWS_FULL + j
        pltpu.sync_copy(src2.at[pl.ds(r, 1)], sidx)
        pltpu.sync_copy(dst2.at[pl.ds(r, 1)], didx)
        pltpu.sync_copy(w2.at[pl.ds(r, 1)], wbuf)

        @pl.loop(0, C // 16)
        def _grp(g):
            sv = sidx[0, pl.ds(g * 16, 16)]
            dv = didx[0, pl.ds(g * 16, 16)]
            wv = wbuf[0, pl.ds(g * 16, 16)]
            wbuf[0, pl.ds(g * 16, 16)] = jnp.where(sv == dv, 0.0, wv)

        pltpu.sync_copy(wbuf.at[0], deg_sh.at[sidx.at[0]], add=True)

    plsc.subcore_barrier()
    pltpu.sync_copy(deg_sh, dinv_v)

    @pl.loop(0, NPAD // 16)
    def _newt(i):
        dinv_v[pl.ds(i * 16, 16)] = _newton_rsqrt(dinv_v[pl.ds(i * 16, 16)])

    @pl.loop(0, ROWS_HALF)
    def _nrm(j):
        r = c * (ROWS // NC) + s * ROWS_HALF + j
        pltpu.sync_copy(src2.at[pl.ds(r, 1)], sidx)
        pltpu.sync_copy(dst2.at[pl.ds(r, 1)], didx)
        pltpu.sync_copy(w2.at[pl.ds(r, 1)], wbuf)

        @pl.loop(0, C // 16)
        def _grp(g):
            sv = sidx[0, pl.ds(g * 16, 16)]
            dv = didx[0, pl.ds(g * 16, 16)]
            wv = wbuf[0, pl.ds(g * 16, 16)]
            wv = jnp.where(sv == dv, 0.0, wv)
            dsv = plsc.load_gather(dinv_v, [sv])
            ddv = plsc.load_gather(dinv_v, [dv])
            nbuf[0, pl.ds(g * 16, 16)] = -(dsv * wv * ddv)

        pltpu.sync_copy(nbuf, norm_out.at[pl.ds(r, 1)])


def _make_norm_kernel():
    mesh = plsc.VectorSubcoreMesh(core_axis_name="c", subcore_axis_name="s")
    return pl.kernel(
        _norm_body,
        out_type=jax.ShapeDtypeStruct((ROWS, C), _f32),
        mesh=mesh,
        scratch_types=[
            pltpu.VMEM((1, C), jnp.int32),
            pltpu.VMEM((1, C), jnp.int32),
            pltpu.VMEM((1, C), _f32),
            pltpu.VMEM((1, C), _f32),
            pltpu.VMEM((NPAD // NS,), _f32),
            pltpu.VMEM((NPAD,), _f32),
            pltpu.VMEM_SHARED((NPAD,), _f32),
            pltpu.SemaphoreType.DMA,
        ],
        compiler_params=pltpu.CompilerParams(needs_layout_passes=False, use_tc_tiling_on_sc=False),
        name="cheb_norm_sc",
    )


def _prop_phase(tab_refs, out_ref, src2, dst2, norm2, sidx, didx, nrm,
                rows, rowsb, wbbuf, acc_sh, sem, s, row_base, nrows):
    @pl.loop(0, WB)
    def _z(rr):
        for cc in range(H // 16):
            wbbuf[rr, pl.ds(cc * 16, 16)] = jnp.zeros((16,), _f32)

    @pl.loop(0, NROW_TILE // WB)
    def _zc(i):
        pltpu.sync_copy(
            wbbuf, acc_sh.at[pl.ds(s * NROW_TILE + i * WB, WB)])

    plsc.subcore_barrier()

    @pl.loop(0, nrows)
    def _chunk(j):
        r = row_base + j
        pltpu.sync_copy(src2.at[pl.ds(r, 1)], sidx)
        pltpu.sync_copy(dst2.at[pl.ds(r, 1)], didx)
        pltpu.sync_copy(norm2.at[pl.ds(r, 1)], nrm)
        cp0 = pltpu.async_copy(tab_refs[0].at[sidx.at[0]], rows, sem)
        if len(tab_refs) > 1:
            cp1 = pltpu.async_copy(tab_refs[1].at[sidx.at[0]], rowsb, sem)
        cp0.wait()
        if len(tab_refs) > 1:
            cp1.wait()

        @pl.loop(0, C // 16)
        def _scale(g):
            nv16 = nrm[0, pl.ds(g * 16, 16)]
            for ll in range(16):
                nv = jnp.full((16,), nv16[ll])
                rr = g * 16 + ll
                for cc in range(H // 16):
                    sl = pl.ds(cc * 16, 16)
                    if len(tab_refs) > 1:
                        rows[rr, sl] = (rows[rr, sl] + rowsb[rr, sl]) * nv
                    else:
                        rows[rr, sl] = rows[rr, sl] * nv

        pltpu.sync_copy(rows, acc_sh.at[didx.at[0]], add=True)

    plsc.subcore_barrier()

    @pl.loop(0, NROW_TILE // WB)
    def _wb(i):
        off = s * NROW_TILE + i * WB
        pltpu.sync_copy(acc_sh.at[pl.ds(off, WB)], wbbuf)
        pltpu.sync_copy(wbbuf, out_ref.at[pl.ds(off, WB)])


def _prop_body(mode, tabA, tabB, src2, dst2, norm2, outA, outB,
               sidx, didx, nrm, rows, rowsb, wbbuf, acc_sh, sem):
    c = lax.axis_index("c")
    s = lax.axis_index("s")
    if mode == "fsplit":
        base = lambda cc: s * ROWS_FULL
        nrows = ROWS_FULL
    else:
        base = lambda cc: cc * (ROWS // NC) + s * ROWS_HALF
        nrows = ROWS_HALF
    tabs0 = (tabA, tabB) if mode == "esplit2" else (
        (tabA,) if mode != "fsplit" else (tabA,))
    tabs1 = (tabA, tabB) if mode == "esplit2" else (
        (tabB,) if mode == "fsplit" else (tabA,))

    @pl.when(c == 0)
    def _c0():
        _prop_phase(tabs0, outA, src2, dst2, norm2, sidx, didx, nrm,
                    rows, rowsb, wbbuf, acc_sh, sem, s, base(0), nrows)

    @pl.when(c == 1)
    def _c1():
        _prop_phase(tabs1, outB, src2, dst2, norm2, sidx, didx, nrm,
                    rows, rowsb, wbbuf, acc_sh, sem, s, base(1), nrows)


def _make_prop_kernel(mode):
    mesh = plsc.VectorSubcoreMesh(core_axis_name="c", subcore_axis_name="s")
    body = functools.partial(_prop_body, mode)
    n_in = 2 if mode in ("fsplit", "esplit2") else 1

    def wrapped(tabA, tabB, src2, dst2, norm2, outA, outB, *scratch):
        body(tabA, tabB, src2, dst2, norm2, outA, outB, *scratch)

    def single(tabA, src2, dst2, norm2, outA, outB, *scratch):
        body(tabA, tabA, src2, dst2, norm2, outA, outB, *scratch)

    return pl.kernel(
        wrapped if n_in == 2 else single,
        out_type=(jax.ShapeDtypeStruct((N, H), _f32),
                  jax.ShapeDtypeStruct((N, H), _f32)),
        mesh=mesh,
        scratch_types=[
            pltpu.VMEM((1, C), jnp.int32),
            pltpu.VMEM((1, C), jnp.int32),
            pltpu.VMEM((1, C), _f32),
            pltpu.VMEM((C, H), _f32),
            pltpu.VMEM((C, H), _f32),
            pltpu.VMEM((WB, H), _f32),
            pltpu.VMEM_SHARED((N, H), _f32),
            pltpu.SemaphoreType.DMA,
        ],
        compiler_params=pltpu.CompilerParams(needs_layout_passes=False, use_tc_tiling_on_sc=False),
        name=f"cheb_prop_{mode}_sc",
    )


def _prep_body(x_ref, w_ref, g1_ref, g2_ref, xw_ref):
    y = jnp.dot(x_ref[...], w_ref[...], preferred_element_type=_f32,
                precision=lax.Precision.HIGHEST)
    g1_ref[...] = y[:, 0:H]
    g2_ref[...] = y[:, H:2 * H]
    xw_ref[...] = y[:, 2 * H:3 * H]


def _conv0_fin_body(xw, a1, cp0, cp1, b, g, bt, out):
    t = xw[...] + a1[...] + 2.0 * (cp0[...] + cp1[...]) + b[...]
    mu = jnp.mean(t, axis=0, keepdims=True)
    var = jnp.mean((t - mu) ** 2, axis=0, keepdims=True)
    out[...] = jnp.maximum(
        (t - mu) * lax.rsqrt(var + 1e-5) * g[...] + bt[...], 0.0)


def _convn_fin_body(h, tp0, tp1, sp0, sp1, w, b, g, bt, out, tbuf):
    hp = lambda a, wslc: jnp.dot(a, wslc, preferred_element_type=_f32,
                                 precision=lax.Precision.HIGHEST)
    blk = N // 10
    for i in range(10):
        sl = pl.ds(i * blk, blk)
        tbuf[sl, :] = (hp(h[sl, :], w[0:H])
                       + hp(tp0[sl, :] + tp1[sl, :], w[H:2 * H])
                       + hp(2.0 * (sp0[sl, :] + sp1[sl, :]), w[2 * H:3 * H])
                       + b[...])
    t = tbuf[...]
    mu = jnp.mean(t, axis=0, keepdims=True)
    var = jnp.mean((t - mu) ** 2, axis=0, keepdims=True)
    out[...] = jnp.maximum(
        (t - mu) * lax.rsqrt(var + 1e-5) * g[...] + bt[...], 0.0)


def _lstm_body(seq_ref, wih0, whh0, b0, wih1, whh1, b1, out):
    nb = seq_ref.shape[1]
    z = jnp.zeros((nb, H), _f32)

    def step(t, carry):
        h1, c1, h2, c2 = carry
        xt = seq_ref[t]
        g1 = (jnp.dot(xt, wih0[...], preferred_element_type=_f32,
                      precision=lax.Precision.HIGHEST)
              + jnp.dot(h1, whh0[...], preferred_element_type=_f32,
                        precision=lax.Precision.HIGHEST) + b0[...])
        i1, f1, gg1, o1 = jnp.split(g1, 4, axis=1)
        c1 = jax.nn.sigmoid(f1) * c1 + jax.nn.sigmoid(i1) * jnp.tanh(gg1)
        h1 = jax.nn.sigmoid(o1) * jnp.tanh(c1)
        g2 = (jnp.dot(h1, wih1[...], preferred_element_type=_f32,
                      precision=lax.Precision.HIGHEST)
              + jnp.dot(h2, whh1[...], preferred_element_type=_f32,
                        precision=lax.Precision.HIGHEST) + b1[...])
        i2, f2, gg2, o2 = jnp.split(g2, 4, axis=1)
        c2 = jax.nn.sigmoid(f2) * c2 + jax.nn.sigmoid(i2) * jnp.tanh(gg2)
        h2 = jax.nn.sigmoid(o2) * jnp.tanh(c2)
        return h1, c1, h2, c2

    _, _, h2, _ = lax.fori_loop(0, SEQ, step, (z, z, z, z))
    out[...] = h2


def _head_body(h3, tf, w1, b1, w2, b2, w3, b3, hz, ar, hout):
    hcat = jnp.concatenate([h3[...], tf[...]], axis=1)
    hout[...] = hcat
    pooled = jnp.mean(hcat, axis=0, keepdims=True)
    hi = lax.Precision.HIGHEST
    z = jnp.maximum(
        jnp.dot(pooled, w1[...], preferred_element_type=_f32, precision=hi)
        + b1[...], 0.0)
    hz[...] = jax.nn.sigmoid(
        jnp.dot(z, w2[...], preferred_element_type=_f32, precision=hi)
        + b2[...])
    ar[...] = jax.nn.sigmoid(
        jnp.dot(z, w3[...], preferred_element_type=_f32, precision=hi)
        + b3[...])


def kernel(x, edge_index, edge_weight, temporal_seq, conv0_W, conv0_b,
           conv1_W, conv1_b, conv2_W, conv2_b, bn_gamma, bn_beta,
           lstm_Wih, lstm_Whh, lstm_bih, lstm_bhh, fc1_W, fc1_b,
           fc2_W, fc2_b, fc3_W, fc3_b):
    src2 = edge_index[0].reshape(ROWS, C).astype(jnp.int32)
    dst2 = edge_index[1].reshape(ROWS, C).astype(jnp.int32)
    w2 = edge_weight.reshape(ROWS, C)

    norm2 = _make_norm_kernel()(src2, dst2, w2)

    wcat0 = jnp.concatenate(
        [conv0_W[1], conv0_W[2], conv0_W[0] - conv0_W[2]], axis=1)
    g1, g2, xw = pl.pallas_call(
        _prep_body,
        out_shape=(jax.ShapeDtypeStruct((N, H), _f32),) * 3,
        grid=(10,),
        in_specs=[pl.BlockSpec((N // 10, F_IN), lambda i: (i, 0)),
                  pl.BlockSpec((F_IN, 3 * H), lambda i: (0, 0))],
        out_specs=(pl.BlockSpec((N // 10, H), lambda i: (i, 0)),) * 3,
        name="conv0_prep_tc",
    )(x, wcat0)

    prop_f = _make_prop_kernel("fsplit")
    prop_e = _make_prop_kernel("esplit")
    prop_e2 = _make_prop_kernel("esplit2")

    a1, b2v = prop_f(g1, g2, src2, dst2, norm2)
    cp0, cp1 = prop_e(b2v, src2, dst2, norm2)

    h1 = pl.pallas_call(
        _conv0_fin_body,
        out_shape=jax.ShapeDtypeStruct((N, H), _f32),
        name="conv0_finish_tc",
    )(xw, a1, cp0, cp1, conv0_b.reshape(1, H),
      bn_gamma[0].reshape(1, H), bn_beta[0].reshape(1, H))

    def conv_n(h, W, b, gam, bet):
        tp0, tp1 = prop_e(h, src2, dst2, norm2)
        sp0, sp1 = prop_e2(tp0, tp1, src2, dst2, norm2)
        wmix = jnp.concatenate([W[0] - W[2], W[1], W[2]], axis=0)
        return pl.pallas_call(
            _convn_fin_body,
            out_shape=jax.ShapeDtypeStruct((N, H), _f32),
            scratch_shapes=[pltpu.VMEM((N, H), _f32)],
            name="convn_finish_tc",
        )(h, tp0, tp1, sp0, sp1, wmix, b.reshape(1, H),
          gam.reshape(1, H), bet.reshape(1, H))

    h2 = conv_n(h1, conv1_W, conv1_b, bn_gamma[1], bn_beta[1])
    h3 = conv_n(h2, conv2_W, conv2_b, bn_gamma[2], bn_beta[2])

    seq_t = jnp.swapaxes(temporal_seq, 0, 1)
    nb = 10
    tf = pl.pallas_call(
        _lstm_body,
        out_shape=jax.ShapeDtypeStruct((N, H), _f32),
        grid=(nb,),
        in_specs=[pl.BlockSpec((SEQ, N // nb, H), lambda i: (0, i, 0)),
                  pl.BlockSpec((H, 4 * H), lambda i: (0, 0)),
                  pl.BlockSpec((H, 4 * H), lambda i: (0, 0)),
                  pl.BlockSpec((1, 4 * H), lambda i: (0, 0)),
                  pl.BlockSpec((H, 4 * H), lambda i: (0, 0)),
                  pl.BlockSpec((H, 4 * H), lambda i: (0, 0)),
                  pl.BlockSpec((1, 4 * H), lambda i: (0, 0))],
        out_specs=pl.BlockSpec((N // nb, H), lambda i: (i, 0)),
        name="lstm2_tc",
    )(seq_t, lstm_Wih[0].T, lstm_Whh[0].T,
      (lstm_bih[0] + lstm_bhh[0]).reshape(1, 4 * H),
      lstm_Wih[1].T, lstm_Whh[1].T,
      (lstm_bih[1] + lstm_bhh[1]).reshape(1, 4 * H))

    hz, ar, hout = pl.pallas_call(
        _head_body,
        out_shape=(jax.ShapeDtypeStruct((1, 6), _f32),
                   jax.ShapeDtypeStruct((1, 1), _f32),
                   jax.ShapeDtypeStruct((N, 2 * H), _f32)),
        name="head_tc",
    )(h3, tf, fc1_W, fc1_b.reshape(1, H), fc2_W, fc2_b.reshape(1, 6),
      fc3_W, fc3_b.reshape(1, 1))

    return hz, ar, hout


# speedup vs baseline: 1.7860x; 1.7860x over previous
import functools

import jax
import jax.numpy as jnp
from jax import lax
from jax.experimental import pallas as pl
from jax.experimental.pallas import tpu as pltpu
from jax.experimental.pallas import tpu_sc as plsc

N = 10000
E = 320000
F_IN = 128
H = 64
SEQ = 24

NC, NS, L = 2, 16, 16
NPAD = 10240
C = 80
ROWS = E // C
ROWS_FULL = ROWS // NS
ROWS_HALF = ROWS // (NC * NS)
NROW_TILE = N // NS
WB = 125
SB = 5

_f32 = jnp.float32


def _newton_rsqrt(x):
    xi = lax.bitcast_convert_type(x, jnp.int32)
    yi = jnp.int32(0x5F3759DF) - lax.shift_right_logical(xi, 1)
    y = lax.bitcast_convert_type(yi, _f32)
    for _ in range(3):
        y = y * (1.5 - 0.5 * x * y * y)
    return jnp.where(x > 0, y, 0.0)


def _norm_body(src2, dst2, w2, norm_out, sidx, didx, wbuf, nbuf, zbuf,
               dinv_v, deg_sh, sem):
    c = lax.axis_index("c")
    s = lax.axis_index("s")
    del sem

    @pl.loop(0, NPAD // NS // 16)
    def _z(i):
        zbuf[pl.ds(i * 16, 16)] = jnp.zeros((16,), _f32)
    pltpu.sync_copy(zbuf, deg_sh.at[pl.ds(s * (NPAD // NS), NPAD // NS)])
    plsc.subcore_barrier()

    @pl.loop(0, ROWS_FULL)
    def _deg(j):
        r = s * ROWS_FULL + j
        pltpu.sync_copy(src2.at[pl.ds(r, 1)], sidx)
        pltpu.sync_copy(dst2.at[pl.ds(r, 1)], didx)
        pltpu.sync_copy(w2.at[pl.ds(r, 1)], wbuf)

        @pl.loop(0, C // 16)
        def _grp(g):
            sv = sidx[0, pl.ds(g * 16, 16)]
            dv = didx[0, pl.ds(g * 16, 16)]
            wv = wbuf[0, pl.ds(g * 16, 16)]
            wbuf[0, pl.ds(g * 16, 16)] = jnp.where(sv == dv, 0.0, wv)

        pltpu.sync_copy(wbuf.at[0], deg_sh.at[sidx.at[0]], add=True)

    plsc.subcore_barrier()
    pltpu.sync_copy(deg_sh, dinv_v)

    @pl.loop(0, NPAD // 16)
    def _newt(i):
        dinv_v[pl.ds(i * 16, 16)] = _newton_rsqrt(dinv_v[pl.ds(i * 16, 16)])

    @pl.loop(0, ROWS_HALF)
    def _nrm(j):
        r = c * (ROWS // NC) + s * ROWS_HALF + j
        pltpu.sync_copy(src2.at[pl.ds(r, 1)], sidx)
        pltpu.sync_copy(dst2.at[pl.ds(r, 1)], didx)
        pltpu.sync_copy(w2.at[pl.ds(r, 1)], wbuf)

        @pl.loop(0, C // 16)
        def _grp(g):
            sv = sidx[0, pl.ds(g * 16, 16)]
            dv = didx[0, pl.ds(g * 16, 16)]
            wv = wbuf[0, pl.ds(g * 16, 16)]
            wv = jnp.where(sv == dv, 0.0, wv)
            dsv = plsc.load_gather(dinv_v, [sv])
            ddv = plsc.load_gather(dinv_v, [dv])
            nbuf[0, pl.ds(g * 16, 16)] = -(dsv * wv * ddv)

        pltpu.sync_copy(nbuf, norm_out.at[pl.ds(r, 1)])


def _make_norm_kernel():
    mesh = plsc.VectorSubcoreMesh(core_axis_name="c", subcore_axis_name="s")
    return pl.kernel(
        _norm_body,
        out_type=jax.ShapeDtypeStruct((ROWS, C), _f32),
        mesh=mesh,
        scratch_types=[
            pltpu.VMEM((1, C), jnp.int32),
            pltpu.VMEM((1, C), jnp.int32),
            pltpu.VMEM((1, C), _f32),
            pltpu.VMEM((1, C), _f32),
            pltpu.VMEM((NPAD // NS,), _f32),
            pltpu.VMEM((NPAD,), _f32),
            pltpu.VMEM_SHARED((NPAD,), _f32),
            pltpu.SemaphoreType.DMA,
        ],
        compiler_params=pltpu.CompilerParams(needs_layout_passes=False, use_tc_tiling_on_sc=False),
        name="cheb_norm_sc",
    )


def _prop_phase(tab_refs, out_ref, src2, dst2, norm2, sidx, didx, nrm,
                bufs_a, bufs_b, wbbuf, acc_sh, sems_a, sems_b,
                s, row_base, nsb):
    @pl.loop(0, WB)
    def _z(rr):
        for cc in range(H // 16):
            wbbuf[rr, pl.ds(cc * 16, 16)] = jnp.zeros((16,), _f32)

    @pl.loop(0, NROW_TILE // WB)
    def _zc(i):
        pltpu.sync_copy(
            wbbuf, acc_sh.at[pl.ds(s * NROW_TILE + i * WB, WB)])

    plsc.subcore_barrier()

    two = len(tab_refs) > 1

    @pl.loop(0, nsb)
    def _sb(q):
        r0 = row_base + q * SB
        pltpu.sync_copy(src2.at[pl.ds(r0, SB)], sidx)
        pltpu.sync_copy(dst2.at[pl.ds(r0, SB)], didx)
        pltpu.sync_copy(norm2.at[pl.ds(r0, SB)], nrm)

        desc = pltpu.async_copy(tab_refs[0].at[sidx.at[0]], bufs_a[0],
                                sems_a[0])
        if two:
            descb = pltpu.async_copy(tab_refs[1].at[sidx.at[0]], bufs_b[0],
                                     sems_b[0])
        for r in range(SB):
            cur = r % 2
            nxt = (r + 1) % 2
            if r + 1 < SB:
                ndesc = pltpu.async_copy(
                    tab_refs[0].at[sidx.at[r + 1]], bufs_a[nxt], sems_a[nxt])
                if two:
                    ndescb = pltpu.async_copy(
                        tab_refs[1].at[sidx.at[r + 1]], bufs_b[nxt],
                        sems_b[nxt])
            desc.wait()
            if two:
                descb.wait()
            rows = bufs_a[cur]
            rowsb = bufs_b[cur]

            @pl.loop(0, C // 16)
            def _scale(g):
                nv16 = nrm[r, pl.ds(g * 16, 16)]
                for ll in range(16):
                    nv = jnp.full((16,), nv16[ll])
                    rr = g * 16 + ll
                    for cc in range(H // 16):
                        sl = pl.ds(cc * 16, 16)
                        if two:
                            rows[rr, sl] = (rows[rr, sl] + rowsb[rr, sl]) * nv
                        else:
                            rows[rr, sl] = rows[rr, sl] * nv

            pltpu.sync_copy(rows, acc_sh.at[didx.at[r]], add=True)
            if r + 1 < SB:
                desc = ndesc
                if two:
                    descb = ndescb

    plsc.subcore_barrier()

    @pl.loop(0, NROW_TILE // WB)
    def _wb(i):
        off = s * NROW_TILE + i * WB
        pltpu.sync_copy(acc_sh.at[pl.ds(off, WB)], wbbuf)
        pltpu.sync_copy(wbbuf, out_ref.at[pl.ds(off, WB)])


def _prop_body(mode, tabA, tabB, src2, dst2, norm2, outA, outB,
               sidx, didx, nrm, ba0, ba1, bb0, bb1, wbbuf, acc_sh,
               sa0, sa1, sb0, sb1):
    c = lax.axis_index("c")
    s = lax.axis_index("s")
    if mode == "fsplit":
        base = lambda cc: s * ROWS_FULL
        nsb = ROWS_FULL // SB
    else:
        base = lambda cc: cc * (ROWS // NC) + s * ROWS_HALF
        nsb = ROWS_HALF // SB
    tabs0 = (tabA, tabB) if mode == "esplit2" else (tabA,)
    tabs1 = ((tabA, tabB) if mode == "esplit2" else
             ((tabB,) if mode == "fsplit" else (tabA,)))

    @pl.when(c == 0)
    def _c0():
        _prop_phase(tabs0, outA, src2, dst2, norm2, sidx, didx, nrm,
                    (ba0, ba1), (bb0, bb1), wbbuf, acc_sh,
                    (sa0, sa1), (sb0, sb1), s, base(0), nsb)

    @pl.when(c == 1)
    def _c1():
        _prop_phase(tabs1, outB, src2, dst2, norm2, sidx, didx, nrm,
                    (ba0, ba1), (bb0, bb1), wbbuf, acc_sh,
                    (sa0, sa1), (sb0, sb1), s, base(1), nsb)


def _make_prop_kernel(mode):
    mesh = plsc.VectorSubcoreMesh(core_axis_name="c", subcore_axis_name="s")
    body = functools.partial(_prop_body, mode)
    n_in = 2 if mode in ("fsplit", "esplit2") else 1

    def wrapped(tabA, tabB, src2, dst2, norm2, outA, outB, *scratch):
        body(tabA, tabB, src2, dst2, norm2, outA, outB, *scratch)

    def single(tabA, src2, dst2, norm2, outA, outB, *scratch):
        body(tabA, tabA, src2, dst2, norm2, outA, outB, *scratch)

    return pl.kernel(
        wrapped if n_in == 2 else single,
        out_type=(jax.ShapeDtypeStruct((N, H), _f32),
                  jax.ShapeDtypeStruct((N, H), _f32)),
        mesh=mesh,
        scratch_types=[
            pltpu.VMEM((SB, C), jnp.int32),
            pltpu.VMEM((SB, C), jnp.int32),
            pltpu.VMEM((SB, C), _f32),
            pltpu.VMEM((C, H), _f32),
            pltpu.VMEM((C, H), _f32),
            pltpu.VMEM((C, H), _f32),
            pltpu.VMEM((C, H), _f32),
            pltpu.VMEM((WB, H), _f32),
            pltpu.VMEM_SHARED((N, H), _f32),
            pltpu.SemaphoreType.DMA,
            pltpu.SemaphoreType.DMA,
            pltpu.SemaphoreType.DMA,
            pltpu.SemaphoreType.DMA,
        ],
        compiler_params=pltpu.CompilerParams(needs_layout_passes=False, use_tc_tiling_on_sc=False),
        name=f"cheb_prop_{mode}_sc",
    )


def _prep_body(x_ref, w_ref, g1_ref, g2_ref, xw_ref):
    y = jnp.dot(x_ref[...], w_ref[...], preferred_element_type=_f32,
                precision=lax.Precision.HIGHEST)
    g1_ref[...] = y[:, 0:H]
    g2_ref[...] = y[:, H:2 * H]
    xw_ref[...] = y[:, 2 * H:3 * H]


def _conv0_fin_body(xw, a1, cp0, cp1, b, g, bt, out):
    t = xw[...] + a1[...] + 2.0 * (cp0[...] + cp1[...]) + b[...]
    mu = jnp.mean(t, axis=0, keepdims=True)
    var = jnp.mean((t - mu) ** 2, axis=0, keepdims=True)
    out[...] = jnp.maximum(
        (t - mu) * lax.rsqrt(var + 1e-5) * g[...] + bt[...], 0.0)


def _convn_fin_body(h, tp0, tp1, sp0, sp1, w, b, g, bt, out, tbuf):
    hp = lambda a, wslc: jnp.dot(a, wslc, preferred_element_type=_f32,
                                 precision=lax.Precision.HIGHEST)
    blk = N // 10
    for i in range(10):
        sl = pl.ds(i * blk, blk)
        tbuf[sl, :] = (hp(h[sl, :], w[0:H])
                       + hp(tp0[sl, :] + tp1[sl, :], w[H:2 * H])
                       + hp(2.0 * (sp0[sl, :] + sp1[sl, :]), w[2 * H:3 * H])
                       + b[...])
    t = tbuf[...]
    mu = jnp.mean(t, axis=0, keepdims=True)
    var = jnp.mean((t - mu) ** 2, axis=0, keepdims=True)
    out[...] = jnp.maximum(
        (t - mu) * lax.rsqrt(var + 1e-5) * g[...] + bt[...], 0.0)


def _lstm_body(seq_ref, wih0, whh0, b0, wih1, whh1, b1, out):
    nb = seq_ref.shape[1]
    z = jnp.zeros((nb, H), _f32)

    def step(t, carry):
        h1, c1, h2, c2 = carry
        xt = seq_ref[t]
        g1 = (jnp.dot(xt, wih0[...], preferred_element_type=_f32)
              + jnp.dot(h1, whh0[...], preferred_element_type=_f32) + b0[...])
        i1, f1, gg1, o1 = jnp.split(g1, 4, axis=1)
        c1 = jax.nn.sigmoid(f1) * c1 + jax.nn.sigmoid(i1) * jnp.tanh(gg1)
        h1 = jax.nn.sigmoid(o1) * jnp.tanh(c1)
        g2 = (jnp.dot(h1, wih1[...], preferred_element_type=_f32)
              + jnp.dot(h2, whh1[...], preferred_element_type=_f32) + b1[...])
        i2, f2, gg2, o2 = jnp.split(g2, 4, axis=1)
        c2 = jax.nn.sigmoid(f2) * c2 + jax.nn.sigmoid(i2) * jnp.tanh(gg2)
        h2 = jax.nn.sigmoid(o2) * jnp.tanh(c2)
        return h1, c1, h2, c2

    _, _, h2, _ = lax.fori_loop(0, SEQ, step, (z, z, z, z))
    out[...] = h2


def _head_body(h3, tf, w1, b1, w2, b2, w3, b3, hz, ar, hout):
    hcat = jnp.concatenate([h3[...], tf[...]], axis=1)
    hout[...] = hcat
    pooled = jnp.mean(hcat, axis=0, keepdims=True)
    hi = lax.Precision.HIGHEST
    z = jnp.maximum(
        jnp.dot(pooled, w1[...], preferred_element_type=_f32, precision=hi)
        + b1[...], 0.0)
    hz[...] = jax.nn.sigmoid(
        jnp.dot(z, w2[...], preferred_element_type=_f32, precision=hi)
        + b2[...])
    ar[...] = jax.nn.sigmoid(
        jnp.dot(z, w3[...], preferred_element_type=_f32, precision=hi)
        + b3[...])


def kernel(x, edge_index, edge_weight, temporal_seq, conv0_W, conv0_b,
           conv1_W, conv1_b, conv2_W, conv2_b, bn_gamma, bn_beta,
           lstm_Wih, lstm_Whh, lstm_bih, lstm_bhh, fc1_W, fc1_b,
           fc2_W, fc2_b, fc3_W, fc3_b):
    src2 = edge_index[0].reshape(ROWS, C).astype(jnp.int32)
    dst2 = edge_index[1].reshape(ROWS, C).astype(jnp.int32)
    w2 = edge_weight.reshape(ROWS, C)

    norm2 = _make_norm_kernel()(src2, dst2, w2)

    wcat0 = jnp.concatenate(
        [conv0_W[1], conv0_W[2], conv0_W[0] - conv0_W[2]], axis=1)
    g1, g2, xw = pl.pallas_call(
        _prep_body,
        out_shape=(jax.ShapeDtypeStruct((N, H), _f32),) * 3,
        grid=(10,),
        in_specs=[pl.BlockSpec((N // 10, F_IN), lambda i: (i, 0)),
                  pl.BlockSpec((F_IN, 3 * H), lambda i: (0, 0))],
        out_specs=(pl.BlockSpec((N // 10, H), lambda i: (i, 0)),) * 3,
        name="conv0_prep_tc",
    )(x, wcat0)

    prop_f = _make_prop_kernel("fsplit")
    prop_e = _make_prop_kernel("esplit")
    prop_e2 = _make_prop_kernel("esplit2")

    a1, b2v = prop_f(g1, g2, src2, dst2, norm2)
    cp0, cp1 = prop_e(b2v, src2, dst2, norm2)

    h1 = pl.pallas_call(
        _conv0_fin_body,
        out_shape=jax.ShapeDtypeStruct((N, H), _f32),
        name="conv0_finish_tc",
    )(xw, a1, cp0, cp1, conv0_b.reshape(1, H),
      bn_gamma[0].reshape(1, H), bn_beta[0].reshape(1, H))

    def conv_n(h, W, b, gam, bet):
        tp0, tp1 = prop_e(h, src2, dst2, norm2)
        sp0, sp1 = prop_e2(tp0, tp1, src2, dst2, norm2)
        wmix = jnp.concatenate([W[0] - W[2], W[1], W[2]], axis=0)
        return pl.pallas_call(
            _convn_fin_body,
            out_shape=jax.ShapeDtypeStruct((N, H), _f32),
            scratch_shapes=[pltpu.VMEM((N, H), _f32)],
            name="convn_finish_tc",
        )(h, tp0, tp1, sp0, sp1, wmix, b.reshape(1, H),
          gam.reshape(1, H), bet.reshape(1, H))

    h2 = conv_n(h1, conv1_W, conv1_b, bn_gamma[1], bn_beta[1])
    h3 = conv_n(h2, conv2_W, conv2_b, bn_gamma[2], bn_beta[2])

    seq_t = jnp.swapaxes(temporal_seq, 0, 1)
    nb = 10
    tf = pl.pallas_call(
        _lstm_body,
        out_shape=jax.ShapeDtypeStruct((N, H), _f32),
        grid=(nb,),
        in_specs=[pl.BlockSpec((SEQ, N // nb, H), lambda i: (0, i, 0)),
                  pl.BlockSpec((H, 4 * H), lambda i: (0, 0)),
                  pl.BlockSpec((H, 4 * H), lambda i: (0, 0)),
                  pl.BlockSpec((1, 4 * H), lambda i: (0, 0)),
                  pl.BlockSpec((H, 4 * H), lambda i: (0, 0)),
                  pl.BlockSpec((H, 4 * H), lambda i: (0, 0)),
                  pl.BlockSpec((1, 4 * H), lambda i: (0, 0))],
        out_specs=pl.BlockSpec((N // nb, H), lambda i: (i, 0)),
        name="lstm2_tc",
    )(seq_t, lstm_Wih[0].T, lstm_Whh[0].T,
      (lstm_bih[0] + lstm_bhh[0]).reshape(1, 4 * H),
      lstm_Wih[1].T, lstm_Whh[1].T,
      (lstm_bih[1] + lstm_bhh[1]).reshape(1, 4 * H))

    hz, ar, hout = pl.pallas_call(
        _head_body,
        out_shape=(jax.ShapeDtypeStruct((1, 6), _f32),
                   jax.ShapeDtypeStruct((1, 1), _f32),
                   jax.ShapeDtypeStruct((N, 2 * H), _f32)),
        name="head_tc",
    )(h3, tf, fc1_W, fc1_b.reshape(1, H), fc2_W, fc2_b.reshape(1, 6),
      fc3_W, fc3_b.reshape(1, 1))

    return hz, ar, hout


# speedup vs baseline: 2.0693x; 1.1586x over previous
import functools

import jax
import jax.numpy as jnp
from jax import lax
from jax.experimental import pallas as pl
from jax.experimental.pallas import tpu as pltpu
from jax.experimental.pallas import tpu_sc as plsc

N = 10000
E = 320000
F_IN = 128
H = 64
SEQ = 24

NC, NS, L = 2, 16, 16
NPAD = 10240
C = 80
ROWS = E // C
ROWS_FULL = ROWS // NS
ROWS_HALF = ROWS // (NC * NS)
NROW_TILE = N // NS
WB = 125
SB = 5

_f32 = jnp.float32


def _newton_rsqrt(x):
    xi = lax.bitcast_convert_type(x, jnp.int32)
    yi = jnp.int32(0x5F3759DF) - lax.shift_right_logical(xi, 1)
    y = lax.bitcast_convert_type(yi, _f32)
    for _ in range(3):
        y = y * (1.5 - 0.5 * x * y * y)
    return jnp.where(x > 0, y, 0.0)


def _norm_body(src2, dst2, w2, norm_out, sidx, didx, wbuf, nbuf, zbuf,
               dinv_v, deg_sh, sem):
    c = lax.axis_index("c")
    s = lax.axis_index("s")
    del sem

    @pl.loop(0, NPAD // NS // 16)
    def _z(i):
        zbuf[pl.ds(i * 16, 16)] = jnp.zeros((16,), _f32)
    pltpu.sync_copy(zbuf, deg_sh.at[pl.ds(s * (NPAD // NS), NPAD // NS)])
    plsc.subcore_barrier()

    @pl.loop(0, ROWS_FULL // SB)
    def _deg(q):
        r0 = s * ROWS_FULL + q * SB
        pltpu.sync_copy(src2.at[pl.ds(r0, SB)], sidx)
        pltpu.sync_copy(dst2.at[pl.ds(r0, SB)], didx)
        pltpu.sync_copy(w2.at[pl.ds(r0, SB)], wbuf)
        for r in range(SB):

            @pl.loop(0, C // 16)
            def _grp(g):
                sv = sidx[r, pl.ds(g * 16, 16)]
                dv = didx[r, pl.ds(g * 16, 16)]
                wv = wbuf[r, pl.ds(g * 16, 16)]
                wbuf[r, pl.ds(g * 16, 16)] = jnp.where(sv == dv, 0.0, wv)

            pltpu.sync_copy(wbuf.at[r], deg_sh.at[sidx.at[r]], add=True)

    plsc.subcore_barrier()
    pltpu.sync_copy(deg_sh, dinv_v)

    @pl.loop(0, NPAD // 16)
    def _newt(i):
        dinv_v[pl.ds(i * 16, 16)] = _newton_rsqrt(dinv_v[pl.ds(i * 16, 16)])

    @pl.loop(0, ROWS_HALF // SB)
    def _nrm(q):
        r0 = c * (ROWS // NC) + s * ROWS_HALF + q * SB
        pltpu.sync_copy(src2.at[pl.ds(r0, SB)], sidx)
        pltpu.sync_copy(dst2.at[pl.ds(r0, SB)], didx)
        pltpu.sync_copy(w2.at[pl.ds(r0, SB)], wbuf)
        for r in range(SB):

            @pl.loop(0, C // 16)
            def _grp(g):
                sv = sidx[r, pl.ds(g * 16, 16)]
                dv = didx[r, pl.ds(g * 16, 16)]
                wv = wbuf[r, pl.ds(g * 16, 16)]
                wv = jnp.where(sv == dv, 0.0, wv)
                dsv = plsc.load_gather(dinv_v, [sv])
                ddv = plsc.load_gather(dinv_v, [dv])
                nbuf[r, pl.ds(g * 16, 16)] = -(dsv * wv * ddv)

        pltpu.sync_copy(nbuf, norm_out.at[pl.ds(r0, SB)])


def _make_norm_kernel():
    mesh = plsc.VectorSubcoreMesh(core_axis_name="c", subcore_axis_name="s")
    return pl.kernel(
        _norm_body,
        out_type=jax.ShapeDtypeStruct((ROWS, C), _f32),
        mesh=mesh,
        scratch_types=[
            pltpu.VMEM((SB, C), jnp.int32),
            pltpu.VMEM((SB, C), jnp.int32),
            pltpu.VMEM((SB, C), _f32),
            pltpu.VMEM((SB, C), _f32),
            pltpu.VMEM((NPAD // NS,), _f32),
            pltpu.VMEM((NPAD,), _f32),
            pltpu.VMEM_SHARED((NPAD,), _f32),
            pltpu.SemaphoreType.DMA,
        ],
        compiler_params=pltpu.CompilerParams(needs_layout_passes=False, use_tc_tiling_on_sc=False),
        name="cheb_norm_sc",
    )


def _prop_phase(tab_refs, out_ref, src2, dst2, norm2, sidx, didx, nrm,
                bufs_a, bufs_b, wbbuf, acc_sh, sems_a, sems_b,
                s, row_base, nsb):
    @pl.loop(0, WB)
    def _z(rr):
        for cc in range(H // 16):
            wbbuf[rr, pl.ds(cc * 16, 16)] = jnp.zeros((16,), _f32)

    @pl.loop(0, NROW_TILE // WB)
    def _zc(i):
        pltpu.sync_copy(
            wbbuf, acc_sh.at[pl.ds(s * NROW_TILE + i * WB, WB)])

    plsc.subcore_barrier()

    two = len(tab_refs) > 1

    @pl.loop(0, nsb)
    def _sb(q):
        r0 = row_base + q * SB
        pltpu.sync_copy(src2.at[pl.ds(r0, SB)], sidx)
        pltpu.sync_copy(dst2.at[pl.ds(r0, SB)], didx)
        pltpu.sync_copy(norm2.at[pl.ds(r0, SB)], nrm)

        desc = pltpu.async_copy(tab_refs[0].at[sidx.at[0]], bufs_a[0],
                                sems_a[0])
        if two:
            descb = pltpu.async_copy(tab_refs[1].at[sidx.at[0]], bufs_b[0],
                                     sems_b[0])
        for r in range(SB):
            cur = r % 2
            nxt = (r + 1) % 2
            if r + 1 < SB:
                ndesc = pltpu.async_copy(
                    tab_refs[0].at[sidx.at[r + 1]], bufs_a[nxt], sems_a[nxt])
                if two:
                    ndescb = pltpu.async_copy(
                        tab_refs[1].at[sidx.at[r + 1]], bufs_b[nxt],
                        sems_b[nxt])
            desc.wait()
            if two:
                descb.wait()
            rows = bufs_a[cur]
            rowsb = bufs_b[cur]

            @pl.loop(0, C // 16)
            def _scale(g):
                nv16 = nrm[r, pl.ds(g * 16, 16)]
                for ll in range(16):
                    nv = jnp.full((16,), nv16[ll])
                    rr = g * 16 + ll
                    for cc in range(H // 16):
                        sl = pl.ds(cc * 16, 16)
                        if two:
                            rows[rr, sl] = (rows[rr, sl] + rowsb[rr, sl]) * nv
                        else:
                            rows[rr, sl] = rows[rr, sl] * nv

            pltpu.sync_copy(rows, acc_sh.at[didx.at[r]], add=True)
            if r + 1 < SB:
                desc = ndesc
                if two:
                    descb = ndescb

    plsc.subcore_barrier()

    @pl.loop(0, NROW_TILE // WB)
    def _wb(i):
        off = s * NROW_TILE + i * WB
        pltpu.sync_copy(acc_sh.at[pl.ds(off, WB)], wbbuf)
        pltpu.sync_copy(wbbuf, out_ref.at[pl.ds(off, WB)])


def _prop_body(mode, tabA, tabB, src2, dst2, norm2, outA, outB,
               sidx, didx, nrm, ba0, ba1, bb0, bb1, wbbuf, acc_sh,
               sa0, sa1, sb0, sb1):
    c = lax.axis_index("c")
    s = lax.axis_index("s")
    if mode == "fsplit":
        base = lambda cc: s * ROWS_FULL
        nsb = ROWS_FULL // SB
    else:
        base = lambda cc: cc * (ROWS // NC) + s * ROWS_HALF
        nsb = ROWS_HALF // SB
    tabs0 = (tabA, tabB) if mode == "esplit2" else (tabA,)
    tabs1 = ((tabA, tabB) if mode == "esplit2" else
             ((tabB,) if mode == "fsplit" else (tabA,)))

    @pl.when(c == 0)
    def _c0():
        _prop_phase(tabs0, outA, src2, dst2, norm2, sidx, didx, nrm,
                    (ba0, ba1), (bb0, bb1), wbbuf, acc_sh,
                    (sa0, sa1), (sb0, sb1), s, base(0), nsb)

    @pl.when(c == 1)
    def _c1():
        _prop_phase(tabs1, outB, src2, dst2, norm2, sidx, didx, nrm,
                    (ba0, ba1), (bb0, bb1), wbbuf, acc_sh,
                    (sa0, sa1), (sb0, sb1), s, base(1), nsb)


def _make_prop_kernel(mode):
    mesh = plsc.VectorSubcoreMesh(core_axis_name="c", subcore_axis_name="s")
    body = functools.partial(_prop_body, mode)
    n_in = 2 if mode in ("fsplit", "esplit2") else 1

    def wrapped(tabA, tabB, src2, dst2, norm2, outA, outB, *scratch):
        body(tabA, tabB, src2, dst2, norm2, outA, outB, *scratch)

    def single(tabA, src2, dst2, norm2, outA, outB, *scratch):
        body(tabA, tabA, src2, dst2, norm2, outA, outB, *scratch)

    return pl.kernel(
        wrapped if n_in == 2 else single,
        out_type=(jax.ShapeDtypeStruct((N, H), _f32),
                  jax.ShapeDtypeStruct((N, H), _f32)),
        mesh=mesh,
        scratch_types=[
            pltpu.VMEM((SB, C), jnp.int32),
            pltpu.VMEM((SB, C), jnp.int32),
            pltpu.VMEM((SB, C), _f32),
            pltpu.VMEM((C, H), _f32),
            pltpu.VMEM((C, H), _f32),
            pltpu.VMEM((C, H), _f32),
            pltpu.VMEM((C, H), _f32),
            pltpu.VMEM((WB, H), _f32),
            pltpu.VMEM_SHARED((N, H), _f32),
            pltpu.SemaphoreType.DMA,
            pltpu.SemaphoreType.DMA,
            pltpu.SemaphoreType.DMA,
            pltpu.SemaphoreType.DMA,
        ],
        compiler_params=pltpu.CompilerParams(needs_layout_passes=False, use_tc_tiling_on_sc=False),
        name=f"cheb_prop_{mode}_sc",
    )


def _prep_body(x_ref, w_ref, g1_ref, g2_ref, xw_ref):
    y = jnp.dot(x_ref[...], w_ref[...], preferred_element_type=_f32,
                precision=lax.Precision.HIGHEST)
    g1_ref[...] = y[:, 0:H]
    g2_ref[...] = y[:, H:2 * H]
    xw_ref[...] = y[:, 2 * H:3 * H]


def _conv0_fin_body(xw, a1, cp0, cp1, b, g, bt, out):
    t = xw[...] + a1[...] + 2.0 * (cp0[...] + cp1[...]) + b[...]
    mu = jnp.mean(t, axis=0, keepdims=True)
    var = jnp.mean((t - mu) ** 2, axis=0, keepdims=True)
    out[...] = jnp.maximum(
        (t - mu) * lax.rsqrt(var + 1e-5) * g[...] + bt[...], 0.0)


def _convn_fin_body(h, tp0, tp1, sp0, sp1, w, b, g, bt, out, tbuf):
    hp = lambda a, wslc: jnp.dot(a, wslc, preferred_element_type=_f32,
                                 precision=lax.Precision.HIGHEST)
    blk = N // 10
    for i in range(10):
        sl = pl.ds(i * blk, blk)
        tbuf[sl, :] = (hp(h[sl, :], w[0:H])
                       + hp(tp0[sl, :] + tp1[sl, :], w[H:2 * H])
                       + hp(2.0 * (sp0[sl, :] + sp1[sl, :]), w[2 * H:3 * H])
                       + b[...])
    t = tbuf[...]
    mu = jnp.mean(t, axis=0, keepdims=True)
    var = jnp.mean((t - mu) ** 2, axis=0, keepdims=True)
    out[...] = jnp.maximum(
        (t - mu) * lax.rsqrt(var + 1e-5) * g[...] + bt[...], 0.0)


def _lstm_body(seq_ref, wih0, whh0, b0, wih1, whh1, b1, out):
    nb = seq_ref.shape[1]
    z = jnp.zeros((nb, H), _f32)

    def step(t, carry):
        h1, c1, h2, c2 = carry
        xt = seq_ref[t]
        g1 = (jnp.dot(xt, wih0[...], preferred_element_type=_f32)
              + jnp.dot(h1, whh0[...], preferred_element_type=_f32) + b0[...])
        i1, f1, gg1, o1 = jnp.split(g1, 4, axis=1)
        c1 = jax.nn.sigmoid(f1) * c1 + jax.nn.sigmoid(i1) * jnp.tanh(gg1)
        h1 = jax.nn.sigmoid(o1) * jnp.tanh(c1)
        g2 = (jnp.dot(h1, wih1[...], preferred_element_type=_f32)
              + jnp.dot(h2, whh1[...], preferred_element_type=_f32) + b1[...])
        i2, f2, gg2, o2 = jnp.split(g2, 4, axis=1)
        c2 = jax.nn.sigmoid(f2) * c2 + jax.nn.sigmoid(i2) * jnp.tanh(gg2)
        h2 = jax.nn.sigmoid(o2) * jnp.tanh(c2)
        return h1, c1, h2, c2

    _, _, h2, _ = lax.fori_loop(0, SEQ, step, (z, z, z, z))
    out[...] = h2


def _head_body(h3, tf, w1, b1, w2, b2, w3, b3, hz, ar, hout):
    hcat = jnp.concatenate([h3[...], tf[...]], axis=1)
    hout[...] = hcat
    pooled = jnp.mean(hcat, axis=0, keepdims=True)
    hi = lax.Precision.HIGHEST
    z = jnp.maximum(
        jnp.dot(pooled, w1[...], preferred_element_type=_f32, precision=hi)
        + b1[...], 0.0)
    hz[...] = jax.nn.sigmoid(
        jnp.dot(z, w2[...], preferred_element_type=_f32, precision=hi)
        + b2[...])
    ar[...] = jax.nn.sigmoid(
        jnp.dot(z, w3[...], preferred_element_type=_f32, precision=hi)
        + b3[...])


def kernel(x, edge_index, edge_weight, temporal_seq, conv0_W, conv0_b,
           conv1_W, conv1_b, conv2_W, conv2_b, bn_gamma, bn_beta,
           lstm_Wih, lstm_Whh, lstm_bih, lstm_bhh, fc1_W, fc1_b,
           fc2_W, fc2_b, fc3_W, fc3_b):
    src2 = edge_index[0].reshape(ROWS, C).astype(jnp.int32)
    dst2 = edge_index[1].reshape(ROWS, C).astype(jnp.int32)
    w2 = edge_weight.reshape(ROWS, C)

    norm2 = _make_norm_kernel()(src2, dst2, w2)

    wcat0 = jnp.concatenate(
        [conv0_W[1], conv0_W[2], conv0_W[0] - conv0_W[2]], axis=1)
    g1, g2, xw = pl.pallas_call(
        _prep_body,
        out_shape=(jax.ShapeDtypeStruct((N, H), _f32),) * 3,
        grid=(10,),
        in_specs=[pl.BlockSpec((N // 10, F_IN), lambda i: (i, 0)),
                  pl.BlockSpec((F_IN, 3 * H), lambda i: (0, 0))],
        out_specs=(pl.BlockSpec((N // 10, H), lambda i: (i, 0)),) * 3,
        name="conv0_prep_tc",
    )(x, wcat0)

    prop_f = _make_prop_kernel("fsplit")
    prop_e = _make_prop_kernel("esplit")
    prop_e2 = _make_prop_kernel("esplit2")

    a1, b2v = prop_f(g1, g2, src2, dst2, norm2)
    cp0, cp1 = prop_e(b2v, src2, dst2, norm2)

    h1 = pl.pallas_call(
        _conv0_fin_body,
        out_shape=jax.ShapeDtypeStruct((N, H), _f32),
        name="conv0_finish_tc",
    )(xw, a1, cp0, cp1, conv0_b.reshape(1, H),
      bn_gamma[0].reshape(1, H), bn_beta[0].reshape(1, H))

    def conv_n(h, W, b, gam, bet):
        tp0, tp1 = prop_e(h, src2, dst2, norm2)
        sp0, sp1 = prop_e2(tp0, tp1, src2, dst2, norm2)
        wmix = jnp.concatenate([W[0] - W[2], W[1], W[2]], axis=0)
        return pl.pallas_call(
            _convn_fin_body,
            out_shape=jax.ShapeDtypeStruct((N, H), _f32),
            scratch_shapes=[pltpu.VMEM((N, H), _f32)],
            name="convn_finish_tc",
        )(h, tp0, tp1, sp0, sp1, wmix, b.reshape(1, H),
          gam.reshape(1, H), bet.reshape(1, H))

    h2 = conv_n(h1, conv1_W, conv1_b, bn_gamma[1], bn_beta[1])
    h3 = conv_n(h2, conv2_W, conv2_b, bn_gamma[2], bn_beta[2])

    seq_t = jnp.swapaxes(temporal_seq, 0, 1)
    nb = 10
    tf = pl.pallas_call(
        _lstm_body,
        out_shape=jax.ShapeDtypeStruct((N, H), _f32),
        grid=(nb,),
        in_specs=[pl.BlockSpec((SEQ, N // nb, H), lambda i: (0, i, 0)),
                  pl.BlockSpec((H, 4 * H), lambda i: (0, 0)),
                  pl.BlockSpec((H, 4 * H), lambda i: (0, 0)),
                  pl.BlockSpec((1, 4 * H), lambda i: (0, 0)),
                  pl.BlockSpec((H, 4 * H), lambda i: (0, 0)),
                  pl.BlockSpec((H, 4 * H), lambda i: (0, 0)),
                  pl.BlockSpec((1, 4 * H), lambda i: (0, 0))],
        out_specs=pl.BlockSpec((N // nb, H), lambda i: (i, 0)),
        name="lstm2_tc",
    )(seq_t, lstm_Wih[0].T, lstm_Whh[0].T,
      (lstm_bih[0] + lstm_bhh[0]).reshape(1, 4 * H),
      lstm_Wih[1].T, lstm_Whh[1].T,
      (lstm_bih[1] + lstm_bhh[1]).reshape(1, 4 * H))

    hz, ar, hout = pl.pallas_call(
        _head_body,
        out_shape=(jax.ShapeDtypeStruct((1, 6), _f32),
                   jax.ShapeDtypeStruct((1, 1), _f32),
                   jax.ShapeDtypeStruct((N, 2 * H), _f32)),
        name="head_tc",
    )(h3, tf, fc1_W, fc1_b.reshape(1, H), fc2_W, fc2_b.reshape(1, 6),
      fc3_W, fc3_b.reshape(1, 1))

    return hz, ar, hout
